# Initial kernel scaffold; baseline (speedup 1.0000x reference)
import jax
import jax.numpy as jnp
from jax.experimental import pallas as pl


def kernel(x, edge_index, edge_attr, u, node_index, edge_graph_index, params):
    raise NotImplementedError("write your pallas kernel here")


# speedup vs baseline: 1.3681x; 1.3681x over previous
import functools

import jax
import jax.numpy as jnp
from jax import lax
from jax.experimental import pallas as pl

_INTERPRET = False

H = 128


def _mm(x, w):
    return lax.dot_general(x, w, (((1,), (1,)), ((), ())),
                           preferred_element_type=jnp.float32)


def _ln(h, g, b):
    mu = jnp.mean(h, axis=-1, keepdims=True)
    v = jnp.mean((h - mu) * (h - mu), axis=-1, keepdims=True)
    return (h - mu) * lax.rsqrt(v + 1e-5) * g + b


def _mlp_body(x_ref, w1_ref, b1_ref, w2_ref, b2_ref, g_ref, b_ref, o_ref,
              *, final_relu):
    h = jnp.maximum(_mm(x_ref[...], w1_ref[...]) + b1_ref[...], 0.0)
    h = _ln(_mm(h, w2_ref[...]) + b2_ref[...], g_ref[...], b_ref[...])
    if final_relu:
        h = jnp.maximum(h, 0.0)
    o_ref[...] = h


def _mlp_block(x, p, block_rows, final_relu=False):
    n, din = x.shape
    grid = n // block_rows
    full = lambda s: pl.BlockSpec(s, lambda i: (0,) * len(s))
    return pl.pallas_call(
        functools.partial(_mlp_body, final_relu=final_relu),
        grid=(grid,),
        in_specs=[
            pl.BlockSpec((block_rows, din), lambda i: (i, 0)),
            full(p["W1"].shape), full((1, H)), full(p["W2"].shape),
            full((1, H)), full((1, H)), full((1, H)),
        ],
        out_specs=pl.BlockSpec((block_rows, H), lambda i: (i, 0)),
        out_shape=jax.ShapeDtypeStruct((n, H), jnp.float32),
        interpret=_INTERPRET,
    )(x, p["W1"], p["b1"].reshape(1, H), p["W2"], p["b2"].reshape(1, H),
      p["g"].reshape(1, H), p["b"].reshape(1, H))


def _prep_body(nd_ref, gd_ref, w1s_ref, w1d_ref, w1mc_ref, w1ge_ref,
               w1gn_ref, t1_ref, t2_ref, ag_ref, bg_ref):
    nd = nd_ref[...]
    t1_ref[...] = _mm(nd, w1s_ref[...])
    t2_ref[...] = jnp.concatenate(
        [_mm(nd, w1d_ref[...]), _mm(nd, w1mc_ref[...])], axis=1)

    @pl.when(pl.program_id(0) == 0)
    def _():
        gd = gd_ref[...]
        ag_ref[...] = _mm(gd, w1ge_ref[...])
        bg_ref[...] = _mm(gd, w1gn_ref[...])


def _prep(nd, gd, w1s, w1d, w1mc, w1ge, w1gn, block_rows):
    n = nd.shape[0]
    grid = n // block_rows
    full = lambda s: pl.BlockSpec(s, lambda i: (0,) * len(s))
    return pl.pallas_call(
        _prep_body,
        grid=(grid,),
        in_specs=[
            pl.BlockSpec((block_rows, H), lambda i: (i, 0)),
            full((16, H)), full((H, H)), full((H, H)), full((H, H)),
            full((H, H)), full((H, H)),
        ],
        out_specs=[
            pl.BlockSpec((block_rows, H), lambda i: (i, 0)),
            pl.BlockSpec((block_rows, 2 * H), lambda i: (i, 0)),
            full((16, H)), full((16, H)),
        ],
        out_shape=[
            jax.ShapeDtypeStruct((n, H), jnp.float32),
            jax.ShapeDtypeStruct((n, 2 * H), jnp.float32),
            jax.ShapeDtypeStruct((16, H), jnp.float32),
            jax.ShapeDtypeStruct((16, H), jnp.float32),
        ],
        interpret=_INTERPRET,
    )(nd, gd, w1s, w1d, w1mc, w1ge, w1gn)


def _edge_body(s1_ref, s2_ref, ea_ref, egi_ref, ag_ref,
               w1e_ref, b1_ref, w2_ref, b2_ref, g_ref, b_ref,
               w1me_ref, b1m_ref, w2m_ref, b2m_ref, gm_ref, bm_ref,
               ea_out_ref, m_ref, eg_ref):
    egi = egi_ref[0, 0, :]
    nb = egi.shape[0]
    oh = (lax.broadcasted_iota(jnp.int32, (nb, 16), 1)
          == egi[:, None]).astype(jnp.float32)
    s2 = s2_ref[...]
    t = (s1_ref[...] + s2[:, :H] + _mm(ea_ref[...], w1e_ref[...])
         + jnp.dot(oh, ag_ref[...], preferred_element_type=jnp.float32)
         + b1_ref[...])
    t = jnp.maximum(t, 0.0)
    ea_new = jnp.maximum(
        _ln(_mm(t, w2_ref[...]) + b2_ref[...], g_ref[...], b_ref[...]), 0.0)
    ea_out_ref[...] = ea_new

    mp = jnp.maximum(s2[:, H:] + _mm(ea_new, w1me_ref[...]) + b1m_ref[...],
                     0.0)
    m_ref[...] = _ln(_mm(mp, w2m_ref[...]) + b2m_ref[...], gm_ref[...],
                     bm_ref[...])

    @pl.when(pl.program_id(0) == 0)
    def _():
        eg_ref[...] = jnp.zeros_like(eg_ref)

    eg_ref[...] += lax.dot_general(oh, ea_new, (((0,), (0,)), ((), ())),
                                   preferred_element_type=jnp.float32)


def _edge(s1, s2, ea, egi3, ag, peu, pn1, block_rows):
    e = ea.shape[0]
    grid = e // block_rows
    full = lambda s: pl.BlockSpec(s, lambda i: (0,) * len(s))
    return pl.pallas_call(
        _edge_body,
        grid=(grid,),
        in_specs=[
            pl.BlockSpec((block_rows, H), lambda i: (i, 0)),
            pl.BlockSpec((block_rows, 2 * H), lambda i: (i, 0)),
            pl.BlockSpec((block_rows, H), lambda i: (i, 0)),
            pl.BlockSpec((1, 1, block_rows), lambda i: (i, 0, 0)),
            full((16, H)),
            full((H, H)), full((1, H)), full((H, H)), full((1, H)),
            full((1, H)), full((1, H)),
            full((H, H)), full((1, H)), full((H, H)), full((1, H)),
            full((1, H)), full((1, H)),
        ],
        out_specs=[
            pl.BlockSpec((block_rows, H), lambda i: (i, 0)),
            pl.BlockSpec((block_rows, H), lambda i: (i, 0)),
            full((16, H)),
        ],
        out_shape=[
            jax.ShapeDtypeStruct((e, H), jnp.float32),
            jax.ShapeDtypeStruct((e, H), jnp.float32),
            jax.ShapeDtypeStruct((16, H), jnp.float32),
        ],
        interpret=_INTERPRET,
    )(s1, s2, ea, egi3, ag,
      peu["W1"][:, 2 * H:3 * H], peu["b1"].reshape(1, H), peu["W2"],
      peu["b2"].reshape(1, H), peu["g"].reshape(1, H), peu["b"].reshape(1, H),
      pn1["W1"][:, H:], pn1["b1"].reshape(1, H), pn1["W2"],
      pn1["b2"].reshape(1, H), pn1["g"].reshape(1, H), pn1["b"].reshape(1, H))


def _node_body(nd_ref, sp_ref, cp_ref, ni_ref, bg_ref,
               w1a_ref, w1b_ref, b1_ref, w2_ref, b2_ref, g_ref, b_ref,
               nd_out_ref, na_ref):
    ni = ni_ref[0, 0, :]
    nb = ni.shape[0]
    oh = (lax.broadcasted_iota(jnp.int32, (nb, 16), 1)
          == ni[:, None]).astype(jnp.float32)
    s = sp_ref[0] + sp_ref[1]
    cnt = jnp.maximum(cp_ref[0, :, 0] + cp_ref[1, :, 0], 1.0)
    agg = s / cnt[:, None]
    nd = nd_ref[...]
    t = jnp.maximum(
        _mm(nd, w1a_ref[...]) + _mm(agg, w1b_ref[...])
        + jnp.dot(oh, bg_ref[...], preferred_element_type=jnp.float32)
        + b1_ref[...], 0.0)
    nd_new = jnp.maximum(
        _ln(_mm(t, w2_ref[...]) + b2_ref[...], g_ref[...], b_ref[...]), 0.0)
    nd_out_ref[...] = nd_new

    @pl.when(pl.program_id(0) == 0)
    def _():
        na_ref[...] = jnp.zeros_like(na_ref)

    na_ref[...] += lax.dot_general(oh, nd_new, (((0,), (0,)), ((), ())),
                                   preferred_element_type=jnp.float32)


def _node(nd, sp, cp, ni3, bg, pn2, block_rows):
    n = nd.shape[0]
    grid = n // block_rows
    full = lambda s: pl.BlockSpec(s, lambda i: (0,) * len(s))
    return pl.pallas_call(
        _node_body,
        grid=(grid,),
        in_specs=[
            pl.BlockSpec((block_rows, H), lambda i: (i, 0)),
            pl.BlockSpec((2, block_rows, H), lambda i: (0, i, 0)),
            pl.BlockSpec((2, block_rows, 8), lambda i: (0, i, 0)),
            pl.BlockSpec((1, 1, block_rows), lambda i: (i, 0, 0)),
            full((16, H)),
            full((H, H)), full((H, H)), full((1, H)), full((H, H)),
            full((1, H)), full((1, H)), full((1, H)),
        ],
        out_specs=[
            pl.BlockSpec((block_rows, H), lambda i: (i, 0)),
            full((16, H)),
        ],
        out_shape=[
            jax.ShapeDtypeStruct((n, H), jnp.float32),
            jax.ShapeDtypeStruct((16, H), jnp.float32),
        ],
        interpret=_INTERPRET,
    )(nd, sp, cp, ni3, bg,
      pn2["W1"][:, :H], pn2["W1"][:, H:2 * H], pn2["b1"].reshape(1, H),
      pn2["W2"], pn2["b2"].reshape(1, H), pn2["g"].reshape(1, H),
      pn2["b"].reshape(1, H))


def _glob_body(gd_ref, na_ref, eg_ref, w1a_ref, w1b_ref, w1c_ref, b1_ref,
               w2_ref, b2_ref, g_ref, b_ref, o_ref):
    t = jnp.maximum(
        _mm(gd_ref[...], w1a_ref[...]) + _mm(na_ref[...], w1b_ref[...])
        + _mm(eg_ref[...], w1c_ref[...]) + b1_ref[...], 0.0)
    o_ref[...] = jnp.maximum(
        _ln(_mm(t, w2_ref[...]) + b2_ref[...], g_ref[...], b_ref[...]), 0.0)


def _glob(gd, na, eg, pg2):
    return pl.pallas_call(
        _glob_body,
        interpret=_INTERPRET,
        out_shape=jax.ShapeDtypeStruct((16, H), jnp.float32),
    )(gd, na, eg, pg2["W1"][:, :H], pg2["W1"][:, H:2 * H],
      pg2["W1"][:, 2 * H:], pg2["b1"].reshape(1, H), pg2["W2"],
      pg2["b2"].reshape(1, H), pg2["g"].reshape(1, H), pg2["b"].reshape(1, H))


def _gru_body(gd_ref, wz0_ref, wn0_ref, wz1_ref, wn1_ref, h_ref):
    gd = gd_ref[...]
    h1 = ((1.0 - jax.nn.sigmoid(_mm(gd, wz0_ref[...])))
          * jnp.tanh(_mm(gd, wn0_ref[...])))
    h2 = ((1.0 - jax.nn.sigmoid(_mm(h1, wz1_ref[...])))
          * jnp.tanh(_mm(h1, wn1_ref[...])))
    h_ref[0] = h1
    h_ref[1] = h2


def _gru(gd, params):
    wih0, wih1 = params["gru_Wih"]
    return pl.pallas_call(
        _gru_body,
        interpret=_INTERPRET,
        out_shape=jax.ShapeDtypeStruct((2, 16, H), jnp.float32),
    )(gd, wih0[H:2 * H], wih0[2 * H:], wih1[H:2 * H], wih1[2 * H:])


def _gather(table, idx):
    return jnp.take(table, idx, axis=0)


def _scatter_counts(row, n):
    ones = jnp.ones((row.shape[0], 8), jnp.float32)
    c = jax.ops.segment_sum(ones, row, num_segments=n)
    return jnp.stack([c, jnp.zeros_like(c)], axis=0)


def _scatter_sum(m, row, n):
    s = jax.ops.segment_sum(m, row, num_segments=n)
    return jnp.stack([s, jnp.zeros_like(s)], axis=0)


def kernel(x, edge_index, edge_attr, u, node_index, edge_graph_index, params):
    n, e = x.shape[0], edge_attr.shape[0]
    row, col = edge_index[0], edge_index[1]
    be, bn = 2000, 1000

    nd = _mlp_block(x, params["ne"], 2000)
    ea = _mlp_block(edge_attr, params["ee"], 4000)
    gd = _mlp_block(u, params["ge"], 16)

    cp = _scatter_counts(row, n)
    egi3 = edge_graph_index.reshape(e // be, 1, be)
    ni3 = node_index.reshape(n // bn, 1, bn)

    peu, pn1, pn2, pg2 = params["eu"], params["n1"], params["n2"], params["g2"]
    w1s = peu["W1"][:, :H]
    w1d = peu["W1"][:, H:2 * H]
    w1ge = peu["W1"][:, 3 * H:]
    w1mc = pn1["W1"][:, :H]
    w1gn = pn2["W1"][:, 2 * H:]

    for _ in range(3):
        t1, t2, ag, bg = _prep(nd, gd, w1s, w1d, w1mc, w1ge, w1gn, bn)
        s1 = _gather(t1, row)
        s2 = _gather(t2, col)
        ea, m, eg = _edge(s1, s2, ea, egi3, ag, peu, pn1, be)
        sp = _scatter_sum(m, row, n)
        nd, na = _node(nd, sp, cp, ni3, bg, pn2, bn)
        gd = _glob(gd, na, eg, pg2)

    hidden = _gru(gd, params)
    return nd, ea, gd, hidden
